# baseline (speedup 1.0000x reference)
import functools

import jax
import jax.numpy as jnp
from jax import lax
from jax.experimental import pallas as pl
from jax.experimental.pallas import tpu as pltpu
from jax.experimental.pallas import tpu_sc as plsc

_VOCAB_SIZES = [100000] * 26
_EMBED = 32
_B, _T, _H = 1024, 20, 26
_VOCAB = 100000

_TOTAL = _B * _T * _H
_NW = 32
_ROW = 128
_ROWS_PER_W = _TOTAL // (_NW * _ROW)
_K = 13
_STEPS = _ROWS_PER_W // _K

_TC = 512
_TGRID = (2600000 + _TC - 1) // _TC
_SROWS = _TGRID * (_TC // 4)


def _t_body(in_ref, out_ref):
    x = in_ref[...]
    y = x.T
    out_ref[:, 0:32] = y[0:128]
    out_ref[:, 32:64] = y[128:256]
    out_ref[:, 64:96] = y[256:384]
    out_ref[:, 96:128] = y[384:512]


_transpose = pl.pallas_call(
    _t_body,
    grid=(_TGRID,),
    in_specs=[pl.BlockSpec((32, _TC), lambda i: (0, i))],
    out_specs=pl.BlockSpec((_TC // 4, 128), lambda i: (i, 0)),
    out_shape=jax.ShapeDtypeStruct((_SROWS, 128), jnp.float32),
)

_mesh = plsc.VectorSubcoreMesh(core_axis_name="c", subcore_axis_name="s")


@functools.partial(
    pl.kernel,
    mesh=_mesh,
    out_type=jax.ShapeDtypeStruct((_TOTAL, _EMBED), jnp.float32),
    scratch_types=[
        pltpu.VMEM((_ROWS_PER_W, _ROW), jnp.int32),
        pltpu.VMEM((_K * _ROW, _EMBED), jnp.float32),
        pltpu.SemaphoreType.DMA,
    ],
    compiler_params=pltpu.CompilerParams(use_tc_tiling_on_sc=False),
)
def _sc_gather(table_hbm, idx_hbm, out_hbm, idx_v, rows_v, sem):
    wid = lax.axis_index("s") * 2 + lax.axis_index("c")
    row0 = wid * _ROWS_PER_W * _ROW
    wrow = wid * _ROWS_PER_W

    pltpu.sync_copy(idx_hbm.at[wid], idx_v)

    def add_offsets(j, carry):
        off = ((wrow + j) * _ROW // (_B * _T)) * _VOCAB
        for v in range(_ROW // 16):
            sl = pl.ds(v * 16, 16)
            t = idx_v[j, sl] + off
            k = (
                lax.shift_left(lax.shift_right_logical(t, 9), 9)
                + lax.shift_left(lax.bitwise_and(t, 127), 2)
                + lax.bitwise_and(lax.shift_right_logical(t, 7), 3)
            )
            idx_v[j, sl] = k
        return carry

    lax.fori_loop(0, _ROWS_PER_W, add_offsets, 0)

    def step(s, carry):
        copies = []
        for k in range(_K):
            j = s * _K + k
            copies.append(
                pltpu.async_copy(
                    table_hbm.at[idx_v.at[j]],
                    rows_v.at[pl.ds(k * _ROW, _ROW)],
                    sem,
                )
            )
        for c in copies:
            c.wait()
        pltpu.sync_copy(
            rows_v, out_hbm.at[pl.ds(row0 + s * (_K * _ROW), _K * _ROW)]
        )
        return carry

    lax.fori_loop(0, _STEPS, step, 0)


def kernel(indices, table):
    scratch = _transpose(table.T)
    scratch32 = scratch.reshape(_SROWS * 4, _EMBED)
    idx_t = jnp.transpose(indices, (2, 1, 0)).astype(jnp.int32)
    idx3 = idx_t.reshape(_NW, _ROWS_PER_W, _ROW)
    out = _sc_gather(scratch32, idx3)
    out4 = out.reshape(_H, _T, _B, _EMBED)
    return jnp.transpose(out4, (2, 1, 0, 3))
